# baseline (device time: 63297 ns/iter reference)
import jax
import jax.numpy as jnp
from jax import lax
from jax.experimental import pallas as pl
from jax.experimental.pallas import tpu as pltpu

N_DEV = 16


def kernel(x, Wp):
    b, h, w, c = x.shape
    n_local = h * w
    n_global = n_local * N_DEV
    n_out = Wp.shape[1]

    x2 = x.reshape(b, n_local, c)

    def body(x_ref, wp_ref, out_ref, stats_ref, comm_ref, send_sems, recv_sems):
        my = lax.axis_index("i")
        left = lax.rem(my + N_DEV - 1, N_DEV)
        right = lax.rem(my + 1, N_DEV)

        barrier_sem = pltpu.get_barrier_semaphore()
        for nbr in (left, right):
            pl.semaphore_signal(
                barrier_sem, inc=1,
                device_id=(nbr,), device_id_type=pl.DeviceIdType.MESH,
            )
        pl.semaphore_wait(barrier_sem, 2)

        parts = []
        for bi in range(b):
            parts.append(jnp.sum(x_ref[bi], axis=0, keepdims=True))
        for bi in range(b):
            parts.append(jnp.sum(x_ref[bi] * x_ref[bi], axis=0, keepdims=True))
        local = jnp.concatenate(parts, axis=0)
        stats_ref[:, :] = local
        comm_ref[0] = local

        for hop in range(N_DEV - 1):
            send_slot = hop % 2
            recv_slot = (hop + 1) % 2
            rdma = pltpu.make_async_remote_copy(
                src_ref=comm_ref.at[send_slot],
                dst_ref=comm_ref.at[recv_slot],
                send_sem=send_sems.at[send_slot],
                recv_sem=recv_sems.at[recv_slot],
                device_id=(right,),
                device_id_type=pl.DeviceIdType.MESH,
            )
            rdma.start()
            rdma.wait()
            stats_ref[:, :] = stats_ref[:, :] + comm_ref[recv_slot]

        eps = 1e-5
        inv_n = 1.0 / float(n_global)
        for bi in range(b):
            mean = stats_ref[bi : bi + 1, :] * inv_n
            ex2 = stats_ref[b + bi : b + bi + 1, :] * inv_n
            var = ex2 - mean * mean
            inv_std = lax.rsqrt(var + eps)
            hb = (x_ref[bi] - mean) * inv_std
            ab = hb * jax.nn.sigmoid(hb)
            out_ref[bi] = jnp.dot(
                ab, wp_ref[:, :], preferred_element_type=jnp.float32
            )

    out = pl.pallas_call(
        body,
        out_shape=jax.ShapeDtypeStruct((b, n_local, n_out), jnp.float32),
        in_specs=[
            pl.BlockSpec(memory_space=pltpu.VMEM),
            pl.BlockSpec(memory_space=pltpu.VMEM),
        ],
        out_specs=pl.BlockSpec(memory_space=pltpu.VMEM),
        scratch_shapes=[
            pltpu.VMEM((2 * b, c), jnp.float32),
            pltpu.VMEM((2, 2 * b, c), jnp.float32),
            pltpu.SemaphoreType.DMA((2,)),
            pltpu.SemaphoreType.DMA((2,)),
        ],
        compiler_params=pltpu.CompilerParams(collective_id=0),
    )(x2, Wp)

    return out.reshape(b, h, w, n_out)


# device time: 43674 ns/iter; 1.4493x vs baseline; 1.4493x over previous
import jax
import jax.numpy as jnp
from jax import lax
from jax.experimental import pallas as pl
from jax.experimental.pallas import tpu as pltpu

N_DEV = 16


def kernel(x, Wp):
    b, h, w, c = x.shape
    n_local = h * w
    n_global = n_local * N_DEV
    n_out = Wp.shape[1]

    x2 = x.reshape(b, n_local, c)

    steps = (1, 2, 4, 8)

    def body(x_ref, wp_ref, out_ref, send_buf, recv_buf, send_sems, recv_sems):
        my = lax.axis_index("i")

        parts = []
        for bi in range(b):
            parts.append(jnp.sum(x_ref[bi], axis=0, keepdims=True))
        for bi in range(b):
            parts.append(jnp.sum(x_ref[bi] * x_ref[bi], axis=0, keepdims=True))
        acc = jnp.concatenate(parts, axis=0)

        barrier_sem = pltpu.get_barrier_semaphore()
        for k in steps:
            pl.semaphore_signal(
                barrier_sem, inc=1,
                device_id=(my ^ k,), device_id_type=pl.DeviceIdType.MESH,
            )
        pl.semaphore_wait(barrier_sem, len(steps))

        for idx, k in enumerate(steps):
            send_buf[idx] = acc
            rdma = pltpu.make_async_remote_copy(
                src_ref=send_buf.at[idx],
                dst_ref=recv_buf.at[idx],
                send_sem=send_sems.at[idx],
                recv_sem=recv_sems.at[idx],
                device_id=(my ^ k,),
                device_id_type=pl.DeviceIdType.MESH,
            )
            rdma.start()
            rdma.wait()
            acc = acc + recv_buf[idx]

        eps = 1e-5
        inv_n = 1.0 / float(n_global)
        for bi in range(b):
            mean = acc[bi : bi + 1, :] * inv_n
            ex2 = acc[b + bi : b + bi + 1, :] * inv_n
            var = ex2 - mean * mean
            inv_std = lax.rsqrt(var + eps)
            hb = (x_ref[bi] - mean) * inv_std
            ab = hb * jax.nn.sigmoid(hb)
            out_ref[bi] = jnp.dot(
                ab, wp_ref[:, :], preferred_element_type=jnp.float32
            )

    out = pl.pallas_call(
        body,
        out_shape=jax.ShapeDtypeStruct((b, n_local, n_out), jnp.float32),
        in_specs=[
            pl.BlockSpec(memory_space=pltpu.VMEM),
            pl.BlockSpec(memory_space=pltpu.VMEM),
        ],
        out_specs=pl.BlockSpec(memory_space=pltpu.VMEM),
        scratch_shapes=[
            pltpu.VMEM((len(steps), 2 * b, c), jnp.float32),
            pltpu.VMEM((len(steps), 2 * b, c), jnp.float32),
            pltpu.SemaphoreType.DMA((len(steps),)),
            pltpu.SemaphoreType.DMA((len(steps),)),
        ],
        compiler_params=pltpu.CompilerParams(collective_id=0),
    )(x2, Wp)

    return out.reshape(b, h, w, n_out)


# device time: 37817 ns/iter; 1.6738x vs baseline; 1.1549x over previous
import jax
import jax.numpy as jnp
from jax import lax
from jax.experimental import pallas as pl
from jax.experimental.pallas import tpu as pltpu

N_DEV = 16
CH = 4


def kernel(x, Wp):
    b, h, w, c = x.shape
    n_local = h * w
    n_global = n_local * N_DEV
    n_out = Wp.shape[1]
    rows = n_local // CH

    x2 = x.reshape(b, n_local, c)

    def body(x_hbm, wp_ref, out_hbm, x_vmem, out_vmem, send_buf, allrecv,
             in_sems, send_sems, recv_sems, out_sems):
        my = lax.axis_index("i")

        barrier_sem = pltpu.get_barrier_semaphore()
        for d in range(1, N_DEV):
            pl.semaphore_signal(
                barrier_sem, inc=1,
                device_id=(lax.rem(my + d, N_DEV),),
                device_id_type=pl.DeviceIdType.MESH,
            )

        in_copies = []
        for bi in range(b):
            for k in range(CH):
                cp = pltpu.make_async_copy(
                    x_hbm.at[bi, pl.ds(k * rows, rows), :],
                    x_vmem.at[bi, pl.ds(k * rows, rows), :],
                    in_sems.at[bi * CH + k],
                )
                cp.start()
                in_copies.append(cp)

        ones_row = jnp.ones((8, rows), jnp.float32)

        def local_stats(bi):
            s_acc = jnp.zeros((8, c), jnp.float32)
            q_acc = jnp.zeros((8, c), jnp.float32)
            for k in range(CH):
                in_copies[bi * CH + k].wait()
                ch = x_vmem[bi, k * rows : (k + 1) * rows, :]
                s_acc = s_acc + jnp.dot(
                    ones_row, ch, preferred_element_type=jnp.float32
                )
                q_acc = q_acc + jnp.dot(
                    ones_row, ch * ch, preferred_element_type=jnp.float32
                )
            return jnp.concatenate(
                [s_acc[0:1, :], q_acc[0:1, :]], axis=0
            )

        def start_exchange(bi):
            rdmas = []
            for d in range(1, N_DEV):
                r = pltpu.make_async_remote_copy(
                    src_ref=send_buf.at[bi],
                    dst_ref=allrecv.at[bi, d],
                    send_sem=send_sems.at[bi * N_DEV + d],
                    recv_sem=recv_sems.at[bi * N_DEV + d],
                    device_id=(lax.rem(my + d, N_DEV),),
                    device_id_type=pl.DeviceIdType.MESH,
                )
                r.start()
                rdmas.append(r)
            return rdmas

        local0 = local_stats(0)
        send_buf[0] = local0
        allrecv[0, 0] = local0
        pl.semaphore_wait(barrier_sem, N_DEV - 1)
        rdmas0 = start_exchange(0)

        local1 = local_stats(1)
        send_buf[1] = local1
        allrecv[1, 0] = local1
        rdmas1 = start_exchange(1)

        eps = 1e-5
        inv_n = 1.0 / float(n_global)
        pending = {}

        def epilogue(bi, rdmas):
            for r in rdmas:
                r.wait()
            total = jnp.sum(allrecv[bi], axis=0)
            mean = total[0:1, :] * inv_n
            var = total[1:2, :] * inv_n - mean * mean
            inv_std = lax.rsqrt(var + eps)
            scale = inv_std
            bias = -mean * inv_std
            for k in range(CH):
                slot = (bi * CH + k) % 2
                if slot in pending:
                    pending[slot].wait()
                ch = x_vmem[bi, k * rows : (k + 1) * rows, :]
                hb = ch * scale + bias
                ab = hb * jax.nn.sigmoid(hb)
                out_vmem[slot] = jnp.dot(
                    ab, wp_ref[:, :], preferred_element_type=jnp.float32
                )
                cp = pltpu.make_async_copy(
                    out_vmem.at[slot],
                    out_hbm.at[bi, pl.ds(k * rows, rows), :],
                    out_sems.at[slot],
                )
                cp.start()
                pending[slot] = cp

        epilogue(0, rdmas0)
        epilogue(1, rdmas1)
        for cp in pending.values():
            cp.wait()

    out = pl.pallas_call(
        body,
        out_shape=jax.ShapeDtypeStruct((b, n_local, n_out), jnp.float32),
        in_specs=[
            pl.BlockSpec(memory_space=pl.ANY),
            pl.BlockSpec(memory_space=pltpu.VMEM),
        ],
        out_specs=pl.BlockSpec(memory_space=pl.ANY),
        scratch_shapes=[
            pltpu.VMEM((b, n_local, c), jnp.float32),
            pltpu.VMEM((2, rows, n_out), jnp.float32),
            pltpu.VMEM((b, 2, c), jnp.float32),
            pltpu.VMEM((b, N_DEV, 2, c), jnp.float32),
            pltpu.SemaphoreType.DMA((b * CH,)),
            pltpu.SemaphoreType.DMA((b * N_DEV,)),
            pltpu.SemaphoreType.DMA((b * N_DEV,)),
            pltpu.SemaphoreType.DMA((2,)),
        ],
        compiler_params=pltpu.CompilerParams(collective_id=0),
    )(x2, Wp)

    return out.reshape(b, h, w, n_out)
